# baseline (speedup 1.0000x reference)
import functools

import jax
import jax.numpy as jnp
from jax import lax
from jax.experimental import pallas as pl
from jax.experimental.pallas import tpu as pltpu
from jax.experimental.pallas import tpu_sc as plsc

N_TOKENS = 4608
D = 768
K = 1024
TB = 576
G = N_TOKENS // TB


def _argmin_body(x_ref, et_ref, idx_ref):
    et = et_ref[...]
    e2 = jnp.sum(et * et, axis=0, keepdims=True)
    scores = lax.dot_general(
        x_ref[...], et, (((1,), (0,)), ((), ())),
        preferred_element_type=jnp.float32,
    )
    d = e2 - 2.0 * scores
    m = jnp.min(d, axis=1, keepdims=True)
    col = lax.broadcasted_iota(jnp.int32, d.shape, 1)
    idx = jnp.min(jnp.where(d == m, col, K), axis=1)
    idx_ref[0, 0, :] = idx.astype(jnp.int32)


def _argmin_indices(x_flat, emb_t):
    out = pl.pallas_call(
        _argmin_body,
        grid=(G,),
        in_specs=[
            pl.BlockSpec((TB, D), lambda i: (i, 0)),
            pl.BlockSpec((D, K), lambda i: (0, 0)),
        ],
        out_specs=pl.BlockSpec((1, 1, TB), lambda i: (i, 0, 0)),
        out_shape=jax.ShapeDtypeStruct((G, 1, TB), jnp.int32),
    )(x_flat, emb_t)
    return out.reshape(N_TOKENS)


def _make_gather():
    info = plsc.get_sparse_core_info()
    nc, ns = info.num_cores, info.num_subcores
    nw = nc * ns
    b_per_w = N_TOKENS // nw
    mesh = plsc.VectorSubcoreMesh(core_axis_name="c", subcore_axis_name="s")

    @functools.partial(
        pl.kernel,
        mesh=mesh,
        out_type=jax.ShapeDtypeStruct((N_TOKENS, D), jnp.float32),
        scratch_types=[
            pltpu.VMEM((b_per_w,), jnp.int32),
            pltpu.VMEM((b_per_w, D), jnp.float32),
            pltpu.SemaphoreType.DMA,
        ],
    )
    def gather(table_hbm, idx_hbm, out_hbm, idx_v, rows_v, sem):
        wid = lax.axis_index("s") * nc + lax.axis_index("c")
        base = wid * b_per_w
        pltpu.sync_copy(idx_hbm.at[pl.ds(base, b_per_w)], idx_v)
        pltpu.async_copy(table_hbm.at[idx_v], rows_v, sem).wait()
        pltpu.sync_copy(rows_v, out_hbm.at[pl.ds(base, b_per_w)])

    return gather


def kernel(x, embedding):
    B, T, _ = x.shape
    x_flat = x.reshape(B * T, D)
    idx = _argmin_indices(x_flat, embedding.T)
    quantized = _make_gather()(embedding, idx)
    return quantized.reshape(B, T, D)
